# baseline (speedup 1.0000x reference)
import functools

import jax
import jax.numpy as jnp
from jax import lax
from jax.experimental import pallas as pl
from jax.experimental.pallas import tpu as pltpu
from jax.experimental.pallas import tpu_sc as plsc

_N = 10000
_D = 128
_E = 320000
_BLK = 1000

_NCORES = 2
_NSUB = 16
_NW = _NCORES * _NSUB
_CH = 128
_NP = 10240
_RPT = _NP // _NSUB
_NCH = 84
_NRGN = 88
_EPT = _NRGN * _CH
_EPW = _E // _NW
_EPAD = _NW * _EPT


def _sc_edge_body(x_hbm, src_hbm, dst_hbm, att_hbm, u_out, s_out,
                  rows0, rows1, srcc0, srcc1, srcc2, dstc0, dstc1, dstc2,
                  attc0, attc1, attc2, u_sh, s_sh,
                  lsem0, lsem1, lsem2, gsem0, gsem1, ssem0, ssem1):
    c = lax.axis_index("c")
    s = lax.axis_index("s")
    w = c * _NSUB + s
    zv = jnp.zeros((16,), jnp.float32)
    rows_b = (rows0, rows1)
    srcc_b = (srcc0, srcc1, srcc2)
    dstc_b = (dstc0, dstc1, dstc2)
    attc_b = (attc0, attc1, attc2)
    gsem_b = (gsem0, gsem1)
    ssem_b = (ssem0, ssem1)
    lsem_b = (lsem0, lsem1, lsem2)
    base = w * _EPT

    def issue_loads(i_chunk, b3):
        off = base + i_chunk * _CH
        pltpu.async_copy(src_hbm.at[pl.ds(off, _CH)], srcc_b[b3],
                         lsem_b[b3])
        pltpu.async_copy(dst_hbm.at[pl.ds(off, _CH)], dstc_b[b3],
                         lsem_b[b3])
        pltpu.async_copy(att_hbm.at[pl.ds(off, _CH)], attc_b[b3],
                         lsem_b[b3])

    def wait_loads(b3):
        pltpu.make_async_copy(src_hbm.at[pl.ds(0, _CH)], srcc_b[b3],
                              lsem_b[b3]).wait()
        pltpu.make_async_copy(dst_hbm.at[pl.ds(0, _CH)], dstc_b[b3],
                              lsem_b[b3]).wait()
        pltpu.make_async_copy(att_hbm.at[pl.ds(0, _CH)], attc_b[b3],
                              lsem_b[b3]).wait()

    def issue_gather(b3, b2):
        pltpu.async_copy(x_hbm.at[srcc_b[b3]], rows_b[b2], gsem_b[b2])

    def wait_gather(b2):
        pltpu.make_async_copy(x_hbm.at[pl.ds(0, _CH)], rows_b[b2],
                              gsem_b[b2]).wait()

    def issue_scatter(b3, b2):
        pltpu.async_copy(rows_b[b2], u_sh.at[dstc_b[b3]], ssem_b[b2],
                         add=True)
        pltpu.sync_copy(attc_b[b3], s_sh.at[dstc_b[b3]], add=True)

    def wait_scatter(b2):
        pltpu.make_async_copy(x_hbm.at[pl.ds(0, _CH)], rows_b[b2],
                              ssem_b[b2]).wait()

    def scale(b3, b2):
        rows_v = rows_b[b2]
        att_c = attc_b[b3]

        def scale_body(g, carry2):
            av = att_c[pl.ds(g * 16, 16)]
            for l in range(16):
                a = av[l]
                k = g * 16 + l
                for j in range(8):
                    sl = pl.ds(j * 16, 16)
                    rows_v[k, sl] = rows_v[k, sl] * a
            return carry2
        lax.fori_loop(0, _CH // 16, scale_body, 0)

    issue_loads(0, 0)
    issue_loads(1, 1)

    def zrow_body(i, carry):
        for j in range(8):
            rows0[i, pl.ds(j * 16, 16)] = zv
        return carry
    lax.fori_loop(0, _CH, zrow_body, 0)

    row0 = s * _RPT
    for t in range(_RPT // _CH):
        pltpu.sync_copy(rows0, u_sh.at[pl.ds(row0 + t * _CH, _CH)])
        pltpu.sync_copy(rows0.at[0], s_sh.at[pl.ds(row0 + t * _CH, _CH)])
    plsc.subcore_barrier()

    wait_loads(0)
    issue_gather(0, 0)

    def body(o, carry):
        for u in range(6):
            i = o * 6 + u
            b2 = u % 2
            b3 = u % 3
            b3p1 = (u + 1) % 3
            b3p2 = (u + 2) % 3
            if u == 0:
                @pl.when(o > 0)
                def _():
                    wait_scatter(1 - b2)
            else:
                wait_scatter(1 - b2)
            issue_loads(i + 2, b3p2)
            wait_loads(b3p1)
            issue_gather(b3p1, 1 - b2)
            wait_gather(b2)
            scale(b3, b2)
            issue_scatter(b3, b2)
        return carry
    lax.fori_loop(0, _NCH // 6, body, 0)
    wait_scatter(1)
    wait_gather(0)
    wait_loads((_NCH + 1) % 3)
    plsc.subcore_barrier()

    pltpu.sync_copy(u_sh.at[pl.ds(row0, _RPT)],
                    u_out.at[c, pl.ds(row0, _RPT)])
    pltpu.sync_copy(s_sh.at[pl.ds(row0, _RPT)],
                    s_out.at[c, pl.ds(row0, _RPT)])


def _sc_edge_pass(x, src, dst, att):
    mesh = plsc.VectorSubcoreMesh(core_axis_name="c", subcore_axis_name="s")
    fn = functools.partial(
        pl.kernel,
        mesh=mesh,
        out_type=[
            jax.ShapeDtypeStruct((_NCORES, _NP, _D), jnp.float32),
            jax.ShapeDtypeStruct((_NCORES, _NP), jnp.float32),
        ],
        scratch_types=[
            pltpu.VMEM((_CH, _D), jnp.float32),
            pltpu.VMEM((_CH, _D), jnp.float32),
            pltpu.VMEM((_CH,), jnp.int32),
            pltpu.VMEM((_CH,), jnp.int32),
            pltpu.VMEM((_CH,), jnp.int32),
            pltpu.VMEM((_CH,), jnp.int32),
            pltpu.VMEM((_CH,), jnp.int32),
            pltpu.VMEM((_CH,), jnp.int32),
            pltpu.VMEM((_CH,), jnp.float32),
            pltpu.VMEM((_CH,), jnp.float32),
            pltpu.VMEM((_CH,), jnp.float32),
            pltpu.VMEM_SHARED((_NP, _D), jnp.float32),
            pltpu.VMEM_SHARED((_NP,), jnp.float32),
            pltpu.SemaphoreType.DMA,
            pltpu.SemaphoreType.DMA,
            pltpu.SemaphoreType.DMA,
            pltpu.SemaphoreType.DMA,
            pltpu.SemaphoreType.DMA,
            pltpu.SemaphoreType.DMA,
            pltpu.SemaphoreType.DMA,
        ],
    )(_sc_edge_body)
    return fn(x, src, dst, att)


def _dense_body(num0_ref, num1_ref, den0_ref, den1_ref, x_ref, w_ref,
                b_ref, g_ref, be_ref, o_ref):
    num = num0_ref[...] + num1_ref[...]
    den = den0_ref[...] + den1_ref[...]
    x = x_ref[...]
    aggr = jnp.where(den > 0.0, num / jnp.where(den > 0.0, den, 1.0), 0.0)
    gelu = 0.5 * aggr * (1.0 + jax.lax.erf(aggr * 0.7071067811865476))
    h = gelu + x
    t = jax.lax.dot_general(h, w_ref[...], (((1,), (1,)), ((), ())),
                            preferred_element_type=jnp.float32)
    t = t + b_ref[...]
    mu = jnp.mean(t, axis=-1, keepdims=True)
    var = jnp.mean((t - mu) ** 2, axis=-1, keepdims=True)
    o_ref[...] = (t - mu) * jax.lax.rsqrt(var + 1e-5) * g_ref[...] + be_ref[...]


def _dense_layer(num0, num1, den0, den1, x, w, b, g, be):
    row_spec = pl.BlockSpec((_BLK, _D), lambda i: (i, 0))
    den_spec = pl.BlockSpec((_BLK, 1), lambda i: (i, 0))
    vec_spec = pl.BlockSpec((1, _D), lambda i: (0, 0))
    return pl.pallas_call(
        _dense_body,
        grid=(_N // _BLK,),
        in_specs=[row_spec, row_spec, den_spec, den_spec, row_spec,
                  pl.BlockSpec((_D, _D), lambda i: (0, 0)),
                  vec_spec, vec_spec, vec_spec],
        out_specs=row_spec,
        out_shape=jax.ShapeDtypeStruct((_N, _D), jnp.float32),
    )(num0, num1, den0, den1, x, w, b, g, be)


def _per_worker_pad(arr, dtype):
    a = arr.reshape(_NW, _EPW)
    padded = jnp.concatenate(
        [a, jnp.zeros((_NW, _EPT - _EPW), dtype)], axis=1)
    return padded.reshape(_EPAD)


def kernel(node_attr, edge_index, batch_idx, adv_atts, W0, b0, g0, be0,
           W1, b1, g1, be1):
    src = _per_worker_pad(edge_index[0], jnp.int32)
    dst = _per_worker_pad(edge_index[1], jnp.int32)
    att0 = _per_worker_pad(adv_atts[0], jnp.float32)
    att1 = _per_worker_pad(adv_atts[1], jnp.float32)

    x = node_attr
    for att, w, b, g, be in ((att0, W0, b0, g0, be0),
                             (att1, W1, b1, g1, be1)):
        u, sden = _sc_edge_pass(x, src, dst, att)
        x = _dense_layer(u[0, :_N], u[1, :_N],
                         sden[0, :_N].reshape(_N, 1),
                         sden[1, :_N].reshape(_N, 1),
                         x, w, b.reshape(1, _D), g.reshape(1, _D),
                         be.reshape(1, _D))
    return x
